# baseline (speedup 1.0000x reference)
import functools

import jax
import jax.numpy as jnp
from jax import lax
from jax.experimental import pallas as pl
from jax.experimental.pallas import tpu as pltpu
from jax.experimental.pallas import tpu_sc as plsc

H, E, NS, NR, K, T = 768, 3072, 2, 8, 2, 2048
S = T * K
BM = 128
P = 5120
NT = P // BM
LANES = 128
NWORK = 32
TPW = T // NWORK
CPW = S // NWORK // 16


def _gelu(v):
    return v * 0.5 * (1.0 + lax.erf(v * 0.7071067811865476))


def _router_body(x_ref, rw_ref, ti_ref, tp_ref):
    lg = jnp.dot(x_ref[...], rw_ref[...].T, preferred_element_type=jnp.float32)
    col = lax.broadcasted_iota(jnp.int32, lg.shape, 1)
    valid = col < NR
    lg = jnp.where(valid, lg, -jnp.inf)
    m = jnp.max(lg, axis=1, keepdims=True)
    el = jnp.where(valid, jnp.exp(lg - m), 0.0)
    probs = el / jnp.sum(el, axis=1, keepdims=True)
    p1 = jnp.max(probs, axis=1, keepdims=True)
    i1 = jnp.min(jnp.where(probs == p1, col, LANES - 1), axis=1, keepdims=True)
    probs2 = jnp.where(col == i1, -1.0, probs)
    p2 = jnp.max(probs2, axis=1, keepdims=True)
    i2 = jnp.min(jnp.where(probs2 == p2, col, LANES - 1), axis=1, keepdims=True)
    ti_ref[...] = jnp.where(col == 0, i1, jnp.where(col == 1, i2, 0))
    tp_ref[...] = jnp.where(col == 0, p1, jnp.where(col == 1, p2, 0.0))


def _dispatch_body(eid_hbm, x_hbm, xs_hbm, pos_hbm, te_hbm,
                   eid_v, rank_v, pos_v, start_v, te_v, pe_v, po_v, xrows_v,
                   sem):
    wid = lax.axis_index("s") * 2 + lax.axis_index("c")
    pltpu.sync_copy(eid_hbm, eid_v)

    def rank_chunk(j, carry):
        v = eid_v[pl.ds(j * 16, 16)]
        rnk = jnp.zeros((16,), jnp.int32)
        out = []
        for e in range(NR):
            mi = (v == e).astype(jnp.int32)
            cs = plsc.cumsum(mi)
            rnk = jnp.where(v == e, carry[e] + cs - 1, rnk)
            out.append(carry[e] + jnp.sum(mi))
        rank_v[pl.ds(j * 16, 16)] = rnk
        return tuple(out)

    counts = lax.fori_loop(0, S // 16, rank_chunk,
                           tuple(jnp.int32(0) for _ in range(NR)))

    starts = []
    acc = jnp.int32(0)
    for e in range(NR):
        starts.append(acc)
        acc = jnp.bitwise_and(acc + counts[e] + (BM - 1), jnp.int32(-BM))
    lane = lax.broadcasted_iota(jnp.int32, (16,), 0)
    svec = jnp.zeros((16,), jnp.int32)
    for e in range(NR):
        svec = jnp.where(lane == e, starts[e], svec)
    start_v[...] = svec

    def pos_chunk(j, _):
        v = eid_v[pl.ds(j * 16, 16)]
        st = plsc.load_gather(start_v, [v])
        pos_v[pl.ds(j * 16, 16)] = st + rank_v[pl.ds(j * 16, 16)]
        return 0

    lax.fori_loop(wid * CPW, (wid + 1) * CPW, pos_chunk, 0)

    @pl.when(wid == 0)
    def _():
        lax.fori_loop(CPW, S // 16, pos_chunk, 0)
        pltpu.sync_copy(pos_v, pos_hbm)
        for j in range(3):
            tv = lax.broadcasted_iota(jnp.int32, (16,), 0) + 16 * j
            tev = jnp.full((16,), -1, jnp.int32)
            for e in range(NR):
                tev = tev + (tv >= starts[e] // BM).astype(jnp.int32)
            te_v[pl.ds(j * 16, 16)] = jnp.minimum(tev, NR - 1)
        pltpu.sync_copy(te_v, te_hbm)

    base = wid * TPW * K
    for j in range(TPW // 16):
        idx2 = 2 * lane + base + 32 * j
        pe_v[pl.ds(j * 16, 16)] = plsc.load_gather(pos_v, [idx2])
        po_v[pl.ds(j * 16, 16)] = plsc.load_gather(pos_v, [idx2 + 1])
    pltpu.sync_copy(x_hbm.at[pl.ds(wid * TPW, TPW)], xrows_v)
    pltpu.async_copy(xrows_v, xs_hbm.at[pe_v], sem).wait()
    pltpu.async_copy(xrows_v, xs_hbm.at[po_v], sem).wait()


BE = 1024
EC = E // BE


def _routed_body(te_ref, xs_ref, up_ref, dn_ref, out_ref, acc_ref):
    c = pl.program_id(0)
    i = pl.program_id(1)
    h = jnp.dot(xs_ref[...].astype(jnp.bfloat16),
                up_ref[0].astype(jnp.bfloat16).T,
                preferred_element_type=jnp.float32)
    g = _gelu(h)
    contrib = jnp.dot(g.astype(jnp.bfloat16),
                      dn_ref[0].astype(jnp.bfloat16).T,
                      preferred_element_type=jnp.float32)
    sl = pl.ds(i * BM, BM)

    @pl.when(c == 0)
    def _():
        acc_ref[sl, :] = contrib

    @pl.when(c > 0)
    def _():
        acc_ref[sl, :] = acc_ref[sl, :] + contrib

    out_ref[...] = acc_ref[sl, :]


def _shared_body(x_ref, up_ref, dn_ref, out_ref, xb_ref):
    e = pl.program_id(0)
    c = pl.program_id(1)

    @pl.when((e == 0) & (c == 0))
    def _():
        out_ref[...] = x_ref[...]
        xb_ref[...] = x_ref[...].astype(jnp.bfloat16)

    h = jnp.dot(xb_ref[...], up_ref[0].astype(jnp.bfloat16).T,
                preferred_element_type=jnp.float32)
    g = _gelu(h)
    out_ref[...] += jnp.dot(g.astype(jnp.bfloat16),
                            dn_ref[0].astype(jnp.bfloat16).T,
                            preferred_element_type=jnp.float32)


def _combine_body(sout_hbm, ys_hbm, pos_hbm, tp_hbm, out_hbm,
                  posc_v, tpc_v, rows_v, sv_v, ov_v,
                  sem_g0, sem_g1, sem_s0, sem_s1, sem_o0, sem_o1):
    wid = lax.axis_index("s") * 2 + lax.axis_index("c")
    base = wid * TPW
    nj = TPW // 16
    sems_g = (sem_g0, sem_g1)
    sems_s = (sem_s0, sem_s1)
    sems_o = (sem_o0, sem_o1)

    pltpu.sync_copy(pos_hbm.at[pl.ds(wid * (TPW // 16), TPW // 16)], posc_v)
    pltpu.sync_copy(tp_hbm.at[pl.ds(base * K, TPW * K)], tpc_v)

    def start_in(j):
        b = j % 2
        g = pltpu.async_copy(ys_hbm.at[posc_v.at[j]],
                             rows_v.at[b], sems_g[b])
        s = pltpu.async_copy(sout_hbm.at[pl.ds(base + 16 * j, 16)],
                             sv_v.at[b], sems_s[b])
        return g, s

    pend = start_in(0)
    out_pend = [None, None]
    for j in range(nj):
        b = j % 2
        g, s = pend
        g.wait()
        s.wait()
        if j + 1 < nj:
            pend = start_in(j + 1)

        def tok(jt, _):
            p0 = plsc.load_gather(tpc_v, [jnp.full((16,), 32 * j, jnp.int32) + 2 * jt])
            p1 = plsc.load_gather(tpc_v, [jnp.full((16,), 32 * j + 1, jnp.int32) + 2 * jt])
            for cc in range(H // 16):
                cs = pl.ds(cc * 16, 16)
                ov_v[b, jt, cs] = (sv_v[b, jt, cs] + p0 * rows_v[b, 2 * jt, cs]
                                   + p1 * rows_v[b, 2 * jt + 1, cs])
            return 0

        lax.fori_loop(0, 16, tok, 0)
        if out_pend[b] is not None:
            out_pend[b].wait()
        out_pend[b] = pltpu.async_copy(ov_v.at[b],
                                       out_hbm.at[pl.ds(base + 16 * j, 16)],
                                       sems_o[b])
    for h in out_pend:
        if h is not None:
            h.wait()


def kernel(x, shared_up, shared_down, routed_up, routed_down, router_w):
    rw_pad = jnp.zeros((LANES, H), jnp.float32).at[:NR].set(router_w)

    ti_wide, tp_wide = pl.pallas_call(
        _router_body,
        grid=(T // 512,),
        in_specs=[
            pl.BlockSpec((512, H), lambda t: (t, 0)),
            pl.BlockSpec((LANES, H), lambda t: (0, 0)),
        ],
        out_specs=[
            pl.BlockSpec((512, LANES), lambda t: (t, 0)),
            pl.BlockSpec((512, LANES), lambda t: (t, 0)),
        ],
        out_shape=[
            jax.ShapeDtypeStruct((T, LANES), jnp.int32),
            jax.ShapeDtypeStruct((T, LANES), jnp.float32),
        ],
    )(x, rw_pad)
    eid = ti_wide[:, :K].reshape(S)
    tp = tp_wide[:, :K].reshape(S)

    mesh = plsc.VectorSubcoreMesh(core_axis_name="c", subcore_axis_name="s")
    xs, pos, te = pl.kernel(
        _dispatch_body,
        out_type=[
            jax.ShapeDtypeStruct((P, H), jnp.float32),
            jax.ShapeDtypeStruct((S,), jnp.int32),
            jax.ShapeDtypeStruct((48,), jnp.int32),
        ],
        mesh=mesh,
        compiler_params=pltpu.CompilerParams(needs_layout_passes=False),
        scratch_types=[
            pltpu.VMEM((S,), jnp.int32),
            pltpu.VMEM((S,), jnp.int32),
            pltpu.VMEM((S,), jnp.int32),
            pltpu.VMEM((16,), jnp.int32),
            pltpu.VMEM((48,), jnp.int32),
            pltpu.VMEM((TPW,), jnp.int32),
            pltpu.VMEM((TPW,), jnp.int32),
            pltpu.VMEM((TPW, H), jnp.float32),
            pltpu.SemaphoreType.DMA,
        ],
    )(eid, x)

    ys = pl.pallas_call(
        _routed_body,
        grid_spec=pltpu.PrefetchScalarGridSpec(
            num_scalar_prefetch=1,
            grid=(EC, NT),
            in_specs=[
                pl.BlockSpec((BM, H), lambda c, i, te_r: (i, 0)),
                pl.BlockSpec((1, BE, H), lambda c, i, te_r: (te_r[i], c, 0)),
                pl.BlockSpec((1, H, BE), lambda c, i, te_r: (te_r[i], 0, c)),
            ],
            out_specs=pl.BlockSpec((BM, H), lambda c, i, te_r: (i, 0)),
            scratch_shapes=[pltpu.VMEM((P, H), jnp.float32)],
        ),
        out_shape=jax.ShapeDtypeStruct((P, H), jnp.float32),
        compiler_params=pltpu.CompilerParams(
            dimension_semantics=("arbitrary", "arbitrary"),
        ),
    )(te[:NT], xs, routed_up, routed_down)

    s_out = pl.pallas_call(
        _shared_body,
        grid=(NS, E // 512),
        in_specs=[
            pl.BlockSpec((T, H), lambda e, c: (0, 0)),
            pl.BlockSpec((1, 512, H), lambda e, c: (e, c, 0)),
            pl.BlockSpec((1, H, 512), lambda e, c: (e, 0, c)),
        ],
        out_specs=pl.BlockSpec((T, H), lambda e, c: (0, 0)),
        out_shape=jax.ShapeDtypeStruct((T, H), jnp.float32),
        scratch_shapes=[pltpu.VMEM((T, H), jnp.bfloat16)],
        compiler_params=pltpu.CompilerParams(
            dimension_semantics=("arbitrary", "arbitrary"),
        ),
    )(x, shared_up, shared_down)

    out = pl.kernel(
        _combine_body,
        out_type=jax.ShapeDtypeStruct((T, H), jnp.float32),
        mesh=mesh,
        compiler_params=pltpu.CompilerParams(needs_layout_passes=False),
        scratch_types=[
            pltpu.VMEM((TPW // 16, 32), jnp.int32),
            pltpu.VMEM((TPW * K,), jnp.float32),
            pltpu.VMEM((2, 32, H), jnp.float32),
            pltpu.VMEM((2, 16, H), jnp.float32),
            pltpu.VMEM((2, 16, H), jnp.float32),
            pltpu.SemaphoreType.DMA,
            pltpu.SemaphoreType.DMA,
            pltpu.SemaphoreType.DMA,
            pltpu.SemaphoreType.DMA,
            pltpu.SemaphoreType.DMA,
            pltpu.SemaphoreType.DMA,
        ],
    )(s_out, ys, pos.reshape(S // 32, 32), tp)
    return out


# speedup vs baseline: 1.2216x; 1.2216x over previous
import functools

import jax
import jax.numpy as jnp
from jax import lax
from jax.experimental import pallas as pl
from jax.experimental.pallas import tpu as pltpu
from jax.experimental.pallas import tpu_sc as plsc

H, E, NS, NR, K, T = 768, 3072, 2, 8, 2, 2048
S = T * K
BM = 256
P = 6144
NT = P // BM
LANES = 128
NWORK = 32
TPW = T // NWORK
CPW = S // NWORK // 16


def _gelu(v):
    return v * 0.5 * (1.0 + lax.erf(v * 0.7071067811865476))


def _router_body(x_ref, rw_ref, ti_ref, tp_ref):
    lg = jnp.dot(x_ref[...], rw_ref[...].T, preferred_element_type=jnp.float32)
    col = lax.broadcasted_iota(jnp.int32, lg.shape, 1)
    valid = col < NR
    lg = jnp.where(valid, lg, -jnp.inf)
    m = jnp.max(lg, axis=1, keepdims=True)
    el = jnp.where(valid, jnp.exp(lg - m), 0.0)
    probs = el / jnp.sum(el, axis=1, keepdims=True)
    p1 = jnp.max(probs, axis=1, keepdims=True)
    i1 = jnp.min(jnp.where(probs == p1, col, LANES - 1), axis=1, keepdims=True)
    probs2 = jnp.where(col == i1, -1.0, probs)
    p2 = jnp.max(probs2, axis=1, keepdims=True)
    i2 = jnp.min(jnp.where(probs2 == p2, col, LANES - 1), axis=1, keepdims=True)
    ti_ref[...] = jnp.where(col == 0, i1, jnp.where(col == 1, i2, 0))
    tp_ref[...] = jnp.where(col == 0, p1, jnp.where(col == 1, p2, 0.0))


def _dispatch_body(eid_hbm, x_hbm, xs_hbm, pos_hbm, te_hbm,
                   eid_v, rank_v, pos_v, start_v, te_v, pe_v, po_v, xrows_v,
                   sem):
    wid = lax.axis_index("s") * 2 + lax.axis_index("c")
    pltpu.sync_copy(eid_hbm, eid_v)

    def rank_chunk(j, carry):
        v = eid_v[pl.ds(j * 16, 16)]
        rnk = jnp.zeros((16,), jnp.int32)
        out = []
        for e in range(NR):
            mi = (v == e).astype(jnp.int32)
            cs = plsc.cumsum(mi)
            rnk = jnp.where(v == e, carry[e] + cs - 1, rnk)
            out.append(carry[e] + jnp.sum(mi))
        rank_v[pl.ds(j * 16, 16)] = rnk
        return tuple(out)

    counts = lax.fori_loop(0, S // 16, rank_chunk,
                           tuple(jnp.int32(0) for _ in range(NR)))

    starts = []
    acc = jnp.int32(0)
    for e in range(NR):
        starts.append(acc)
        acc = jnp.bitwise_and(acc + counts[e] + (BM - 1), jnp.int32(-BM))
    lane = lax.broadcasted_iota(jnp.int32, (16,), 0)
    svec = jnp.zeros((16,), jnp.int32)
    for e in range(NR):
        svec = jnp.where(lane == e, starts[e], svec)
    start_v[...] = svec

    def pos_chunk(j, _):
        v = eid_v[pl.ds(j * 16, 16)]
        st = plsc.load_gather(start_v, [v])
        pos_v[pl.ds(j * 16, 16)] = st + rank_v[pl.ds(j * 16, 16)]
        return 0

    lax.fori_loop(wid * CPW, (wid + 1) * CPW, pos_chunk, 0)

    @pl.when(wid == 0)
    def _():
        lax.fori_loop(CPW, S // 16, pos_chunk, 0)
        pltpu.sync_copy(pos_v, pos_hbm)
        for j in range(3):
            tv = lax.broadcasted_iota(jnp.int32, (16,), 0) + 16 * j
            tev = jnp.full((16,), -1, jnp.int32)
            for e in range(NR):
                tev = tev + (tv >= starts[e] // BM).astype(jnp.int32)
            te_v[pl.ds(j * 16, 16)] = jnp.minimum(tev, NR - 1)
        pltpu.sync_copy(te_v, te_hbm)

    base = wid * TPW * K
    for j in range(TPW // 16):
        idx2 = 2 * lane + base + 32 * j
        pe_v[pl.ds(j * 16, 16)] = plsc.load_gather(pos_v, [idx2])
        po_v[pl.ds(j * 16, 16)] = plsc.load_gather(pos_v, [idx2 + 1])
    pltpu.sync_copy(x_hbm.at[pl.ds(wid * TPW, TPW)], xrows_v)
    pltpu.async_copy(xrows_v, xs_hbm.at[pe_v], sem).wait()
    pltpu.async_copy(xrows_v, xs_hbm.at[po_v], sem).wait()


BE = 1024
EC = E // BE


def _routed_body(te_ref, xs_ref, up_ref, dn_ref, out_ref, acc_ref):
    c = pl.program_id(0)
    i = pl.program_id(1)
    h = jnp.dot(xs_ref[...].astype(jnp.bfloat16),
                up_ref[0].astype(jnp.bfloat16).T,
                preferred_element_type=jnp.float32)
    g = _gelu(h)
    contrib = jnp.dot(g.astype(jnp.bfloat16),
                      dn_ref[0].astype(jnp.bfloat16).T,
                      preferred_element_type=jnp.float32)
    sl = pl.ds(i * BM, BM)

    @pl.when(c == 0)
    def _():
        acc_ref[sl, :] = contrib

    @pl.when(c > 0)
    def _():
        acc_ref[sl, :] = acc_ref[sl, :] + contrib

    out_ref[...] = acc_ref[sl, :]


def _shared_body(x_ref, up_ref, dn_ref, out_ref, xb_ref):
    e = pl.program_id(0)
    c = pl.program_id(1)

    @pl.when((e == 0) & (c == 0))
    def _():
        out_ref[...] = x_ref[...]
        xb_ref[...] = x_ref[...].astype(jnp.bfloat16)

    h = jnp.dot(xb_ref[...], up_ref[0].astype(jnp.bfloat16).T,
                preferred_element_type=jnp.float32)
    g = _gelu(h)
    out_ref[...] += jnp.dot(g.astype(jnp.bfloat16),
                            dn_ref[0].astype(jnp.bfloat16).T,
                            preferred_element_type=jnp.float32)


def _combine_body(sout_hbm, ys_hbm, pos_hbm, tp_hbm, out_hbm,
                  posc_v, tpc_v, rows_v, sv_v, ov_v,
                  sem_g0, sem_g1, sem_s0, sem_s1, sem_o0, sem_o1):
    wid = lax.axis_index("s") * 2 + lax.axis_index("c")
    base = wid * TPW
    nj = TPW // 16
    sems_g = (sem_g0, sem_g1)
    sems_s = (sem_s0, sem_s1)
    sems_o = (sem_o0, sem_o1)

    pltpu.sync_copy(pos_hbm.at[pl.ds(wid * (TPW // 16), TPW // 16)], posc_v)
    pltpu.sync_copy(tp_hbm.at[pl.ds(base * K, TPW * K)], tpc_v)

    def start_in(j):
        b = j % 2
        g = pltpu.async_copy(ys_hbm.at[posc_v.at[j]],
                             rows_v.at[b], sems_g[b])
        s = pltpu.async_copy(sout_hbm.at[pl.ds(base + 16 * j, 16)],
                             sv_v.at[b], sems_s[b])
        return g, s

    pend = start_in(0)
    out_pend = [None, None]
    for j in range(nj):
        b = j % 2
        g, s = pend
        g.wait()
        s.wait()
        if j + 1 < nj:
            pend = start_in(j + 1)

        def tok(jt, _):
            p0 = plsc.load_gather(tpc_v, [jnp.full((16,), 32 * j, jnp.int32) + 2 * jt])
            p1 = plsc.load_gather(tpc_v, [jnp.full((16,), 32 * j + 1, jnp.int32) + 2 * jt])
            for cc in range(H // 16):
                cs = pl.ds(cc * 16, 16)
                ov_v[b, jt, cs] = (sv_v[b, jt, cs] + p0 * rows_v[b, 2 * jt, cs]
                                   + p1 * rows_v[b, 2 * jt + 1, cs])
            return 0

        lax.fori_loop(0, 16, tok, 0)
        if out_pend[b] is not None:
            out_pend[b].wait()
        out_pend[b] = pltpu.async_copy(ov_v.at[b],
                                       out_hbm.at[pl.ds(base + 16 * j, 16)],
                                       sems_o[b])
    for h in out_pend:
        if h is not None:
            h.wait()


def kernel(x, shared_up, shared_down, routed_up, routed_down, router_w):
    rw_pad = jnp.zeros((LANES, H), jnp.float32).at[:NR].set(router_w)

    ti_wide, tp_wide = pl.pallas_call(
        _router_body,
        grid=(T // 512,),
        in_specs=[
            pl.BlockSpec((512, H), lambda t: (t, 0)),
            pl.BlockSpec((LANES, H), lambda t: (0, 0)),
        ],
        out_specs=[
            pl.BlockSpec((512, LANES), lambda t: (t, 0)),
            pl.BlockSpec((512, LANES), lambda t: (t, 0)),
        ],
        out_shape=[
            jax.ShapeDtypeStruct((T, LANES), jnp.int32),
            jax.ShapeDtypeStruct((T, LANES), jnp.float32),
        ],
    )(x, rw_pad)
    eid = ti_wide[:, :K].reshape(S)
    tp = tp_wide[:, :K].reshape(S)

    mesh = plsc.VectorSubcoreMesh(core_axis_name="c", subcore_axis_name="s")
    xs, pos, te = pl.kernel(
        _dispatch_body,
        out_type=[
            jax.ShapeDtypeStruct((P, H), jnp.float32),
            jax.ShapeDtypeStruct((S,), jnp.int32),
            jax.ShapeDtypeStruct((48,), jnp.int32),
        ],
        mesh=mesh,
        compiler_params=pltpu.CompilerParams(needs_layout_passes=False),
        scratch_types=[
            pltpu.VMEM((S,), jnp.int32),
            pltpu.VMEM((S,), jnp.int32),
            pltpu.VMEM((S,), jnp.int32),
            pltpu.VMEM((16,), jnp.int32),
            pltpu.VMEM((48,), jnp.int32),
            pltpu.VMEM((TPW,), jnp.int32),
            pltpu.VMEM((TPW,), jnp.int32),
            pltpu.VMEM((TPW, H), jnp.float32),
            pltpu.SemaphoreType.DMA,
        ],
    )(eid, x)

    ys = pl.pallas_call(
        _routed_body,
        grid_spec=pltpu.PrefetchScalarGridSpec(
            num_scalar_prefetch=1,
            grid=(EC, NT),
            in_specs=[
                pl.BlockSpec((BM, H), lambda c, i, te_r: (i, 0)),
                pl.BlockSpec((1, BE, H), lambda c, i, te_r: (te_r[i], c, 0)),
                pl.BlockSpec((1, H, BE), lambda c, i, te_r: (te_r[i], 0, c)),
            ],
            out_specs=pl.BlockSpec((BM, H), lambda c, i, te_r: (i, 0)),
            scratch_shapes=[pltpu.VMEM((P, H), jnp.float32)],
        ),
        out_shape=jax.ShapeDtypeStruct((P, H), jnp.float32),
        compiler_params=pltpu.CompilerParams(
            dimension_semantics=("arbitrary", "arbitrary"),
        ),
    )(te[:NT], xs, routed_up, routed_down)

    s_out = pl.pallas_call(
        _shared_body,
        grid=(NS, E // 512),
        in_specs=[
            pl.BlockSpec((T, H), lambda e, c: (0, 0)),
            pl.BlockSpec((1, 512, H), lambda e, c: (e, c, 0)),
            pl.BlockSpec((1, H, 512), lambda e, c: (e, 0, c)),
        ],
        out_specs=pl.BlockSpec((T, H), lambda e, c: (0, 0)),
        out_shape=jax.ShapeDtypeStruct((T, H), jnp.float32),
        scratch_shapes=[pltpu.VMEM((T, H), jnp.bfloat16)],
        compiler_params=pltpu.CompilerParams(
            dimension_semantics=("arbitrary", "arbitrary"),
        ),
    )(x, shared_up, shared_down)

    out = pl.kernel(
        _combine_body,
        out_type=jax.ShapeDtypeStruct((T, H), jnp.float32),
        mesh=mesh,
        compiler_params=pltpu.CompilerParams(needs_layout_passes=False),
        scratch_types=[
            pltpu.VMEM((TPW // 16, 32), jnp.int32),
            pltpu.VMEM((TPW * K,), jnp.float32),
            pltpu.VMEM((2, 32, H), jnp.float32),
            pltpu.VMEM((2, 16, H), jnp.float32),
            pltpu.VMEM((2, 16, H), jnp.float32),
            pltpu.SemaphoreType.DMA,
            pltpu.SemaphoreType.DMA,
            pltpu.SemaphoreType.DMA,
            pltpu.SemaphoreType.DMA,
            pltpu.SemaphoreType.DMA,
            pltpu.SemaphoreType.DMA,
        ],
    )(s_out, ys, pos.reshape(S // 32, 32), tp)
    return out


# speedup vs baseline: 1.3358x; 1.0935x over previous
import functools

import jax
import jax.numpy as jnp
from jax import lax
from jax.experimental import pallas as pl
from jax.experimental.pallas import tpu as pltpu
from jax.experimental.pallas import tpu_sc as plsc

H, E, NS, NR, K, T = 768, 3072, 2, 8, 2, 2048
S = T * K
BM = 256
P = 6144
NT = P // BM
LANES = 128
NWORK = 32
TPW = T // NWORK
CPW = S // NWORK // 16


def _gelu(v):
    return v * 0.5 * (1.0 + lax.erf(v * 0.7071067811865476))


def _router_body(x_ref, rw_ref, ti_ref, tp_ref):
    lg = jnp.dot(x_ref[...], rw_ref[...].T, preferred_element_type=jnp.float32)
    col = lax.broadcasted_iota(jnp.int32, lg.shape, 1)
    valid = col < NR
    lg = jnp.where(valid, lg, -jnp.inf)
    m = jnp.max(lg, axis=1, keepdims=True)
    el = jnp.where(valid, jnp.exp(lg - m), 0.0)
    probs = el / jnp.sum(el, axis=1, keepdims=True)
    p1 = jnp.max(probs, axis=1, keepdims=True)
    i1 = jnp.min(jnp.where(probs == p1, col, LANES - 1), axis=1, keepdims=True)
    probs2 = jnp.where(col == i1, -1.0, probs)
    p2 = jnp.max(probs2, axis=1, keepdims=True)
    i2 = jnp.min(jnp.where(probs2 == p2, col, LANES - 1), axis=1, keepdims=True)
    ti_ref[...] = jnp.where(col == 0, i1, jnp.where(col == 1, i2, 0))
    tp_ref[...] = jnp.where(col == 0, p1, jnp.where(col == 1, p2, 0.0))


def _dispatch_body(eid_hbm, x_hbm, xs_hbm, pos_hbm, te_hbm,
                   eid_v, rank_v, pos_v, start_v, te_v, pe_v, po_v, xrows_v,
                   sem):
    wid = lax.axis_index("s") * 2 + lax.axis_index("c")
    pltpu.sync_copy(eid_hbm, eid_v)

    def rank_chunk(j, carry):
        v = eid_v[pl.ds(j * 16, 16)]
        rnk = jnp.zeros((16,), jnp.int32)
        out = []
        for e in range(NR):
            mi = (v == e).astype(jnp.int32)
            cs = plsc.cumsum(mi)
            rnk = jnp.where(v == e, carry[e] + cs - 1, rnk)
            out.append(carry[e] + jnp.sum(mi))
        rank_v[pl.ds(j * 16, 16)] = rnk
        return tuple(out)

    counts = lax.fori_loop(0, S // 16, rank_chunk,
                           tuple(jnp.int32(0) for _ in range(NR)))

    starts = []
    acc = jnp.int32(0)
    for e in range(NR):
        starts.append(acc)
        acc = jnp.bitwise_and(acc + counts[e] + (BM - 1), jnp.int32(-BM))
    lane = lax.broadcasted_iota(jnp.int32, (16,), 0)
    svec = jnp.zeros((16,), jnp.int32)
    for e in range(NR):
        svec = jnp.where(lane == e, starts[e], svec)
    start_v[...] = svec

    def pos_chunk(j, _):
        v = eid_v[pl.ds(j * 16, 16)]
        st = plsc.load_gather(start_v, [v])
        pos_v[pl.ds(j * 16, 16)] = st + rank_v[pl.ds(j * 16, 16)]
        return 0

    lax.fori_loop(wid * CPW, (wid + 1) * CPW, pos_chunk, 0)

    @pl.when(wid == 0)
    def _():
        lax.fori_loop(CPW, S // 16, pos_chunk, 0)
        pltpu.sync_copy(pos_v, pos_hbm)
        for j in range(3):
            tv = lax.broadcasted_iota(jnp.int32, (16,), 0) + 16 * j
            tev = jnp.full((16,), -1, jnp.int32)
            for e in range(NR):
                tev = tev + (tv >= starts[e] // BM).astype(jnp.int32)
            te_v[pl.ds(j * 16, 16)] = jnp.minimum(tev, NR - 1)
        pltpu.sync_copy(te_v, te_hbm)

    base = wid * TPW * K
    for j in range(TPW // 16):
        idx2 = 2 * lane + base + 32 * j
        pe_v[pl.ds(j * 16, 16)] = plsc.load_gather(pos_v, [idx2])
        po_v[pl.ds(j * 16, 16)] = plsc.load_gather(pos_v, [idx2 + 1])
    pltpu.sync_copy(x_hbm.at[pl.ds(wid * TPW, TPW)], xrows_v)
    pltpu.async_copy(xrows_v, xs_hbm.at[pe_v], sem).wait()
    pltpu.async_copy(xrows_v, xs_hbm.at[po_v], sem).wait()


BE = 1536
EC = E // BE


def _routed_body(te_ref, xs_ref, up_ref, dn_ref, out_ref, acc_ref):
    c = pl.program_id(0)
    i = pl.program_id(1)
    h = jnp.dot(xs_ref[...].astype(jnp.bfloat16),
                up_ref[0].astype(jnp.bfloat16).T,
                preferred_element_type=jnp.float32)
    g = _gelu(h)
    contrib = jnp.dot(g.astype(jnp.bfloat16),
                      dn_ref[0].astype(jnp.bfloat16).T,
                      preferred_element_type=jnp.float32)
    sl = pl.ds(i * BM, BM)

    @pl.when(c == 0)
    def _():
        acc_ref[sl, :] = contrib

    @pl.when(c > 0)
    def _():
        acc_ref[sl, :] = acc_ref[sl, :] + contrib

    out_ref[...] = acc_ref[sl, :]


def _shared_body(x_ref, up_ref, dn_ref, out_ref, xb_ref):
    e = pl.program_id(0)
    c = pl.program_id(1)

    @pl.when((e == 0) & (c == 0))
    def _():
        out_ref[...] = x_ref[...]
        xb_ref[...] = x_ref[...].astype(jnp.bfloat16)

    h = jnp.dot(xb_ref[...], up_ref[0].astype(jnp.bfloat16).T,
                preferred_element_type=jnp.float32)
    g = _gelu(h)
    out_ref[...] += jnp.dot(g.astype(jnp.bfloat16),
                            dn_ref[0].astype(jnp.bfloat16).T,
                            preferred_element_type=jnp.float32)


def _combine_body(sout_hbm, ys_hbm, pos_hbm, tp_hbm, out_hbm,
                  posc_v, tpc_v, rows_v, sv_v, ov_v,
                  sem_g0, sem_g1, sem_s0, sem_s1, sem_o0, sem_o1):
    wid = lax.axis_index("s") * 2 + lax.axis_index("c")
    base = wid * TPW
    nj = TPW // 16
    sems_g = (sem_g0, sem_g1)
    sems_s = (sem_s0, sem_s1)
    sems_o = (sem_o0, sem_o1)

    pltpu.sync_copy(pos_hbm.at[pl.ds(wid * (TPW // 16), TPW // 16)], posc_v)
    pltpu.sync_copy(tp_hbm.at[pl.ds(base * K, TPW * K)], tpc_v)

    def start_in(j):
        b = j % 2
        g = pltpu.async_copy(ys_hbm.at[posc_v.at[j]],
                             rows_v.at[b], sems_g[b])
        s = pltpu.async_copy(sout_hbm.at[pl.ds(base + 16 * j, 16)],
                             sv_v.at[b], sems_s[b])
        return g, s

    pend = start_in(0)
    out_pend = [None, None]
    for j in range(nj):
        b = j % 2
        g, s = pend
        g.wait()
        s.wait()
        if j + 1 < nj:
            pend = start_in(j + 1)

        def tok(jt, _):
            p0 = plsc.load_gather(tpc_v, [jnp.full((16,), 32 * j, jnp.int32) + 2 * jt])
            p1 = plsc.load_gather(tpc_v, [jnp.full((16,), 32 * j + 1, jnp.int32) + 2 * jt])
            for cc in range(H // 16):
                cs = pl.ds(cc * 16, 16)
                ov_v[b, jt, cs] = (sv_v[b, jt, cs] + p0 * rows_v[b, 2 * jt, cs]
                                   + p1 * rows_v[b, 2 * jt + 1, cs])
            return 0

        lax.fori_loop(0, 16, tok, 0)
        if out_pend[b] is not None:
            out_pend[b].wait()
        out_pend[b] = pltpu.async_copy(ov_v.at[b],
                                       out_hbm.at[pl.ds(base + 16 * j, 16)],
                                       sems_o[b])
    for h in out_pend:
        if h is not None:
            h.wait()


def kernel(x, shared_up, shared_down, routed_up, routed_down, router_w):
    rw_pad = jnp.zeros((LANES, H), jnp.float32).at[:NR].set(router_w)

    ti_wide, tp_wide = pl.pallas_call(
        _router_body,
        grid=(T // 512,),
        in_specs=[
            pl.BlockSpec((512, H), lambda t: (t, 0)),
            pl.BlockSpec((LANES, H), lambda t: (0, 0)),
        ],
        out_specs=[
            pl.BlockSpec((512, LANES), lambda t: (t, 0)),
            pl.BlockSpec((512, LANES), lambda t: (t, 0)),
        ],
        out_shape=[
            jax.ShapeDtypeStruct((T, LANES), jnp.int32),
            jax.ShapeDtypeStruct((T, LANES), jnp.float32),
        ],
    )(x, rw_pad)
    eid = ti_wide[:, :K].reshape(S)
    tp = tp_wide[:, :K].reshape(S)

    mesh = plsc.VectorSubcoreMesh(core_axis_name="c", subcore_axis_name="s")
    xs, pos, te = pl.kernel(
        _dispatch_body,
        out_type=[
            jax.ShapeDtypeStruct((P, H), jnp.float32),
            jax.ShapeDtypeStruct((S,), jnp.int32),
            jax.ShapeDtypeStruct((48,), jnp.int32),
        ],
        mesh=mesh,
        compiler_params=pltpu.CompilerParams(needs_layout_passes=False),
        scratch_types=[
            pltpu.VMEM((S,), jnp.int32),
            pltpu.VMEM((S,), jnp.int32),
            pltpu.VMEM((S,), jnp.int32),
            pltpu.VMEM((16,), jnp.int32),
            pltpu.VMEM((48,), jnp.int32),
            pltpu.VMEM((TPW,), jnp.int32),
            pltpu.VMEM((TPW,), jnp.int32),
            pltpu.VMEM((TPW, H), jnp.float32),
            pltpu.SemaphoreType.DMA,
        ],
    )(eid, x)

    ys = pl.pallas_call(
        _routed_body,
        grid_spec=pltpu.PrefetchScalarGridSpec(
            num_scalar_prefetch=1,
            grid=(EC, NT),
            in_specs=[
                pl.BlockSpec((BM, H), lambda c, i, te_r: (i, 0)),
                pl.BlockSpec((1, BE, H), lambda c, i, te_r: (te_r[i], c, 0)),
                pl.BlockSpec((1, H, BE), lambda c, i, te_r: (te_r[i], 0, c)),
            ],
            out_specs=pl.BlockSpec((BM, H), lambda c, i, te_r: (i, 0)),
            scratch_shapes=[pltpu.VMEM((P, H), jnp.float32)],
        ),
        out_shape=jax.ShapeDtypeStruct((P, H), jnp.float32),
        compiler_params=pltpu.CompilerParams(
            dimension_semantics=("arbitrary", "arbitrary"),
        ),
    )(te[:NT], xs, routed_up, routed_down)

    s_out = pl.pallas_call(
        _shared_body,
        grid=(NS, E // 512),
        in_specs=[
            pl.BlockSpec((T, H), lambda e, c: (0, 0)),
            pl.BlockSpec((1, 512, H), lambda e, c: (e, c, 0)),
            pl.BlockSpec((1, H, 512), lambda e, c: (e, 0, c)),
        ],
        out_specs=pl.BlockSpec((T, H), lambda e, c: (0, 0)),
        out_shape=jax.ShapeDtypeStruct((T, H), jnp.float32),
        scratch_shapes=[pltpu.VMEM((T, H), jnp.bfloat16)],
        compiler_params=pltpu.CompilerParams(
            dimension_semantics=("arbitrary", "arbitrary"),
        ),
    )(x, shared_up, shared_down)

    out = pl.kernel(
        _combine_body,
        out_type=jax.ShapeDtypeStruct((T, H), jnp.float32),
        mesh=mesh,
        compiler_params=pltpu.CompilerParams(needs_layout_passes=False),
        scratch_types=[
            pltpu.VMEM((TPW // 16, 32), jnp.int32),
            pltpu.VMEM((TPW * K,), jnp.float32),
            pltpu.VMEM((2, 32, H), jnp.float32),
            pltpu.VMEM((2, 16, H), jnp.float32),
            pltpu.VMEM((2, 16, H), jnp.float32),
            pltpu.SemaphoreType.DMA,
            pltpu.SemaphoreType.DMA,
            pltpu.SemaphoreType.DMA,
            pltpu.SemaphoreType.DMA,
            pltpu.SemaphoreType.DMA,
            pltpu.SemaphoreType.DMA,
        ],
    )(s_out, ys, pos.reshape(S // 32, 32), tp)
    return out


# speedup vs baseline: 1.3409x; 1.0038x over previous
import functools

import jax
import jax.numpy as jnp
from jax import lax
from jax.experimental import pallas as pl
from jax.experimental.pallas import tpu as pltpu
from jax.experimental.pallas import tpu_sc as plsc

H, E, NS, NR, K, T = 768, 3072, 2, 8, 2, 2048
S = T * K
BM = 256
P = 6144
NT = P // BM
LANES = 128
NWORK = 32
TPW = T // NWORK
CPW = S // NWORK // 16


def _gelu(v):
    return v * 0.5 * (1.0 + lax.erf(v * 0.7071067811865476))


def _router_body(x_ref, rw_ref, ti_ref, tp_ref):
    lg = jnp.dot(x_ref[...], rw_ref[...].T, preferred_element_type=jnp.float32)
    col = lax.broadcasted_iota(jnp.int32, lg.shape, 1)
    valid = col < NR
    lg = jnp.where(valid, lg, -jnp.inf)
    m = jnp.max(lg, axis=1, keepdims=True)
    el = jnp.where(valid, jnp.exp(lg - m), 0.0)
    probs = el / jnp.sum(el, axis=1, keepdims=True)
    p1 = jnp.max(probs, axis=1, keepdims=True)
    i1 = jnp.min(jnp.where(probs == p1, col, LANES - 1), axis=1, keepdims=True)
    probs2 = jnp.where(col == i1, -1.0, probs)
    p2 = jnp.max(probs2, axis=1, keepdims=True)
    i2 = jnp.min(jnp.where(probs2 == p2, col, LANES - 1), axis=1, keepdims=True)
    ti_ref[...] = jnp.where(col == 0, i1, jnp.where(col == 1, i2, 0))
    tp_ref[...] = jnp.where(col == 0, p1, jnp.where(col == 1, p2, 0.0))


def _dispatch_body(eid_hbm, x_hbm, xs_hbm, pos_hbm, te_hbm,
                   eid_v, rank_v, pos_v, start_v, te_v, pe_v, po_v, xrows_v,
                   sem):
    wid = lax.axis_index("s") * 2 + lax.axis_index("c")
    pltpu.sync_copy(eid_hbm, eid_v)

    def rank_chunk(j, carry):
        v = eid_v[pl.ds(j * 16, 16)]
        rnk = jnp.zeros((16,), jnp.int32)
        out = []
        for e in range(NR):
            mi = (v == e).astype(jnp.int32)
            cs = plsc.cumsum(mi)
            rnk = jnp.where(v == e, carry[e] + cs - 1, rnk)
            out.append(carry[e] + jnp.sum(mi))
        rank_v[pl.ds(j * 16, 16)] = rnk
        return tuple(out)

    counts = lax.fori_loop(0, S // 16, rank_chunk,
                           tuple(jnp.int32(0) for _ in range(NR)))

    starts = []
    acc = jnp.int32(0)
    for e in range(NR):
        starts.append(acc)
        acc = jnp.bitwise_and(acc + counts[e] + (BM - 1), jnp.int32(-BM))
    lane = lax.broadcasted_iota(jnp.int32, (16,), 0)
    svec = jnp.zeros((16,), jnp.int32)
    for e in range(NR):
        svec = jnp.where(lane == e, starts[e], svec)
    start_v[...] = svec

    def pos_chunk(j, _):
        v = eid_v[pl.ds(j * 16, 16)]
        st = plsc.load_gather(start_v, [v])
        pos_v[pl.ds(j * 16, 16)] = st + rank_v[pl.ds(j * 16, 16)]
        return 0

    lax.fori_loop(wid * CPW, (wid + 1) * CPW, pos_chunk, 0)

    @pl.when(wid == 0)
    def _():
        lax.fori_loop(CPW, S // 16, pos_chunk, 0)
        pltpu.sync_copy(pos_v, pos_hbm)
        for j in range(3):
            tv = lax.broadcasted_iota(jnp.int32, (16,), 0) + 16 * j
            tev = jnp.full((16,), -1, jnp.int32)
            for e in range(NR):
                tev = tev + (tv >= starts[e] // BM).astype(jnp.int32)
            te_v[pl.ds(j * 16, 16)] = jnp.minimum(tev, NR - 1)
        pltpu.sync_copy(te_v, te_hbm)

    base = wid * TPW * K
    for j in range(TPW // 16):
        idx2 = 2 * lane + base + 32 * j
        pe_v[pl.ds(j * 16, 16)] = plsc.load_gather(pos_v, [idx2])
        po_v[pl.ds(j * 16, 16)] = plsc.load_gather(pos_v, [idx2 + 1])
    pltpu.sync_copy(x_hbm.at[pl.ds(wid * TPW, TPW)], xrows_v)
    pltpu.async_copy(xrows_v, xs_hbm.at[pe_v], sem).wait()
    pltpu.async_copy(xrows_v, xs_hbm.at[po_v], sem).wait()


BE = 1536
EC = E // BE


def _routed_body(te_ref, xs_ref, up_ref, dn_ref, out_ref, acc_ref):
    c = pl.program_id(0)
    i = pl.program_id(1)
    h = jnp.dot(xs_ref[...].astype(jnp.bfloat16),
                up_ref[0].astype(jnp.bfloat16).T,
                preferred_element_type=jnp.float32)
    g = _gelu(h)
    contrib = jnp.dot(g.astype(jnp.bfloat16),
                      dn_ref[0].astype(jnp.bfloat16).T,
                      preferred_element_type=jnp.float32)
    sl = pl.ds(i * BM, BM)

    @pl.when(c == 0)
    def _():
        acc_ref[sl, :] = contrib

    @pl.when(c > 0)
    def _():
        acc_ref[sl, :] = acc_ref[sl, :] + contrib

    out_ref[...] = acc_ref[sl, :]


def _shared_body(x_ref, up_ref, dn_ref, out_ref, xb_ref):
    e = pl.program_id(0)
    c = pl.program_id(1)

    @pl.when((e == 0) & (c == 0))
    def _():
        out_ref[...] = x_ref[...]
        xb_ref[...] = x_ref[...].astype(jnp.bfloat16)

    h = jnp.dot(xb_ref[...], up_ref[0].astype(jnp.bfloat16).T,
                preferred_element_type=jnp.float32)
    g = _gelu(h)
    out_ref[...] += jnp.dot(g.astype(jnp.bfloat16),
                            dn_ref[0].astype(jnp.bfloat16).T,
                            preferred_element_type=jnp.float32)


def _combine_body(sout_hbm, ys_hbm, pos_hbm, tp_hbm, out_hbm,
                  posc_v, tpc_v, rows_v, sv_v, ov_v,
                  sem_g0, sem_g1, sem_s0, sem_s1, sem_o0, sem_o1):
    wid = lax.axis_index("s") * 2 + lax.axis_index("c")
    base = wid * TPW
    nj = TPW // 16
    sems_g = (sem_g0, sem_g1)
    sems_s = (sem_s0, sem_s1)
    sems_o = (sem_o0, sem_o1)

    pltpu.sync_copy(pos_hbm.at[pl.ds(wid * (TPW // 16), TPW // 16)], posc_v)
    pltpu.sync_copy(tp_hbm.at[pl.ds(base * K, TPW * K)], tpc_v)

    def start_in(j):
        b = j % 2
        g = pltpu.async_copy(ys_hbm.at[posc_v.at[j]],
                             rows_v.at[b], sems_g[b])
        s = pltpu.async_copy(sout_hbm.at[pl.ds(base + 16 * j, 16)],
                             sv_v.at[b], sems_s[b])
        return g, s

    pend = start_in(0)
    out_pend = [None, None]
    for j in range(nj):
        b = j % 2
        g, s = pend
        g.wait()
        s.wait()
        if j + 1 < nj:
            pend = start_in(j + 1)

        def tok(jt, _):
            p0 = plsc.load_gather(tpc_v, [jnp.full((16,), 32 * j, jnp.int32) + 2 * jt])
            p1 = plsc.load_gather(tpc_v, [jnp.full((16,), 32 * j + 1, jnp.int32) + 2 * jt])
            for cc in range(H // 16):
                cs = pl.ds(cc * 16, 16)
                ov_v[b, jt, cs] = (sv_v[b, jt, cs] + p0 * rows_v[b, 2 * jt, cs]
                                   + p1 * rows_v[b, 2 * jt + 1, cs])
            return 0

        lax.fori_loop(0, 16, tok, 0)
        if out_pend[b] is not None:
            out_pend[b].wait()
        out_pend[b] = pltpu.async_copy(ov_v.at[b],
                                       out_hbm.at[pl.ds(base + 16 * j, 16)],
                                       sems_o[b])
    for h in out_pend:
        if h is not None:
            h.wait()


def kernel(x, shared_up, shared_down, routed_up, routed_down, router_w):
    rw_pad = jnp.zeros((LANES, H), jnp.float32).at[:NR].set(router_w)

    ti_wide, tp_wide = pl.pallas_call(
        _router_body,
        grid=(T // 512,),
        in_specs=[
            pl.BlockSpec((512, H), lambda t: (t, 0)),
            pl.BlockSpec((LANES, H), lambda t: (0, 0)),
        ],
        out_specs=[
            pl.BlockSpec((512, LANES), lambda t: (t, 0)),
            pl.BlockSpec((512, LANES), lambda t: (t, 0)),
        ],
        out_shape=[
            jax.ShapeDtypeStruct((T, LANES), jnp.int32),
            jax.ShapeDtypeStruct((T, LANES), jnp.float32),
        ],
    )(x, rw_pad)
    eid = ti_wide[:, :K].reshape(S)
    tp = tp_wide[:, :K].reshape(S)

    mesh = plsc.VectorSubcoreMesh(core_axis_name="c", subcore_axis_name="s")
    xs, pos, te = pl.kernel(
        _dispatch_body,
        out_type=[
            jax.ShapeDtypeStruct((P, H), jnp.float32),
            jax.ShapeDtypeStruct((S,), jnp.int32),
            jax.ShapeDtypeStruct((48,), jnp.int32),
        ],
        mesh=mesh,
        compiler_params=pltpu.CompilerParams(needs_layout_passes=False),
        scratch_types=[
            pltpu.VMEM((S,), jnp.int32),
            pltpu.VMEM((S,), jnp.int32),
            pltpu.VMEM((S,), jnp.int32),
            pltpu.VMEM((16,), jnp.int32),
            pltpu.VMEM((48,), jnp.int32),
            pltpu.VMEM((TPW,), jnp.int32),
            pltpu.VMEM((TPW,), jnp.int32),
            pltpu.VMEM((TPW, H), jnp.float32),
            pltpu.SemaphoreType.DMA,
        ],
    )(eid, x)

    ys = pl.pallas_call(
        _routed_body,
        grid_spec=pltpu.PrefetchScalarGridSpec(
            num_scalar_prefetch=1,
            grid=(EC, NT),
            in_specs=[
                pl.BlockSpec((BM, H), lambda c, i, te_r: (i, 0)),
                pl.BlockSpec((1, BE, H), lambda c, i, te_r: (te_r[i], c, 0)),
                pl.BlockSpec((1, H, BE), lambda c, i, te_r: (te_r[i], 0, c)),
            ],
            out_specs=pl.BlockSpec((BM, H), lambda c, i, te_r: (i, 0)),
            scratch_shapes=[pltpu.VMEM((P, H), jnp.float32)],
        ),
        out_shape=jax.ShapeDtypeStruct((P, H), jnp.float32),
        compiler_params=pltpu.CompilerParams(
            dimension_semantics=("arbitrary", "arbitrary"),
        ),
    )(te[:NT], xs, routed_up, routed_down)

    s_out = pl.pallas_call(
        _shared_body,
        grid=(NS, E // 1024),
        in_specs=[
            pl.BlockSpec((T, H), lambda e, c: (0, 0)),
            pl.BlockSpec((1, 1024, H), lambda e, c: (e, c, 0)),
            pl.BlockSpec((1, H, 1024), lambda e, c: (e, 0, c)),
        ],
        out_specs=pl.BlockSpec((T, H), lambda e, c: (0, 0)),
        out_shape=jax.ShapeDtypeStruct((T, H), jnp.float32),
        scratch_shapes=[pltpu.VMEM((T, H), jnp.bfloat16)],
        compiler_params=pltpu.CompilerParams(
            dimension_semantics=("arbitrary", "arbitrary"),
        ),
    )(x, shared_up, shared_down)

    out = pl.kernel(
        _combine_body,
        out_type=jax.ShapeDtypeStruct((T, H), jnp.float32),
        mesh=mesh,
        compiler_params=pltpu.CompilerParams(needs_layout_passes=False),
        scratch_types=[
            pltpu.VMEM((TPW // 16, 32), jnp.int32),
            pltpu.VMEM((TPW * K,), jnp.float32),
            pltpu.VMEM((2, 32, H), jnp.float32),
            pltpu.VMEM((2, 16, H), jnp.float32),
            pltpu.VMEM((2, 16, H), jnp.float32),
            pltpu.SemaphoreType.DMA,
            pltpu.SemaphoreType.DMA,
            pltpu.SemaphoreType.DMA,
            pltpu.SemaphoreType.DMA,
            pltpu.SemaphoreType.DMA,
            pltpu.SemaphoreType.DMA,
        ],
    )(s_out, ys, pos.reshape(S // 32, 32), tp)
    return out


# speedup vs baseline: 1.5046x; 1.1221x over previous
import functools

import jax
import jax.numpy as jnp
from jax import lax
from jax.experimental import pallas as pl
from jax.experimental.pallas import tpu as pltpu
from jax.experimental.pallas import tpu_sc as plsc

H, E, NS, NR, K, T = 768, 3072, 2, 8, 2, 2048
S = T * K
BM = 256
P = 6144
NT = P // BM
LANES = 128
NWORK = 32
TPW = T // NWORK
CPW = S // NWORK // 16


def _gelu(v):
    return v * 0.5 * (1.0 + lax.erf(v * 0.7071067811865476))


def _router_body(x_ref, rw_ref, ti_ref, tp_ref):
    lg = jnp.dot(x_ref[...], rw_ref[...].T, preferred_element_type=jnp.float32)
    col = lax.broadcasted_iota(jnp.int32, lg.shape, 1)
    valid = col < NR
    lg = jnp.where(valid, lg, -jnp.inf)
    m = jnp.max(lg, axis=1, keepdims=True)
    el = jnp.where(valid, jnp.exp(lg - m), 0.0)
    probs = el / jnp.sum(el, axis=1, keepdims=True)
    p1 = jnp.max(probs, axis=1, keepdims=True)
    i1 = jnp.min(jnp.where(probs == p1, col, LANES - 1), axis=1, keepdims=True)
    probs2 = jnp.where(col == i1, -1.0, probs)
    p2 = jnp.max(probs2, axis=1, keepdims=True)
    i2 = jnp.min(jnp.where(probs2 == p2, col, LANES - 1), axis=1, keepdims=True)
    ti_ref[...] = jnp.where(col == 0, i1, jnp.where(col == 1, i2, 0))
    tp_ref[...] = jnp.where(col == 0, p1, jnp.where(col == 1, p2, 0.0))


def _dispatch_body(eid_hbm, x_hbm, xs_hbm, pos_hbm, te_hbm,
                   eid_v, rank_v, pos_v, start_v, te_v, pe_v, po_v, xrows_v,
                   sem):
    wid = lax.axis_index("s") * 2 + lax.axis_index("c")
    pltpu.sync_copy(eid_hbm, eid_v)

    def rank_chunk(j, carry):
        v = eid_v[pl.ds(j * 16, 16)]
        rnk = jnp.zeros((16,), jnp.int32)
        out = []
        for e in range(NR):
            mi = (v == e).astype(jnp.int32)
            cs = plsc.cumsum(mi)
            rnk = jnp.where(v == e, carry[e] + cs - 1, rnk)
            out.append(carry[e] + jnp.sum(mi))
        rank_v[pl.ds(j * 16, 16)] = rnk
        return tuple(out)

    counts = lax.fori_loop(0, S // 16, rank_chunk,
                           tuple(jnp.int32(0) for _ in range(NR)))

    starts = []
    acc = jnp.int32(0)
    for e in range(NR):
        starts.append(acc)
        acc = jnp.bitwise_and(acc + counts[e] + (BM - 1), jnp.int32(-BM))
    lane = lax.broadcasted_iota(jnp.int32, (16,), 0)
    svec = jnp.zeros((16,), jnp.int32)
    for e in range(NR):
        svec = jnp.where(lane == e, starts[e], svec)
    start_v[...] = svec

    def pos_chunk(j, _):
        v = eid_v[pl.ds(j * 16, 16)]
        st = plsc.load_gather(start_v, [v])
        pos_v[pl.ds(j * 16, 16)] = st + rank_v[pl.ds(j * 16, 16)]
        return 0

    lax.fori_loop(wid * CPW, (wid + 1) * CPW, pos_chunk, 0)

    @pl.when(wid == 0)
    def _():
        lax.fori_loop(CPW, S // 16, pos_chunk, 0)
        pltpu.sync_copy(pos_v, pos_hbm)
        for j in range(3):
            tv = lax.broadcasted_iota(jnp.int32, (16,), 0) + 16 * j
            tev = jnp.full((16,), -1, jnp.int32)
            for e in range(NR):
                tev = tev + (tv >= starts[e] // BM).astype(jnp.int32)
            te_v[pl.ds(j * 16, 16)] = jnp.minimum(tev, NR - 1)
        pltpu.sync_copy(te_v, te_hbm)

    base = wid * TPW * K
    for j in range(TPW // 16):
        idx2 = 2 * lane + base + 32 * j
        pe_v[pl.ds(j * 16, 16)] = plsc.load_gather(pos_v, [idx2])
        po_v[pl.ds(j * 16, 16)] = plsc.load_gather(pos_v, [idx2 + 1])
    pltpu.sync_copy(x_hbm.at[pl.ds(wid * TPW, TPW)], xrows_v)
    pltpu.async_copy(xrows_v, xs_hbm.at[pe_v], sem).wait()
    pltpu.async_copy(xrows_v, xs_hbm.at[po_v], sem).wait()


BE = 3072
EC = E // BE


def _routed_body(te_ref, xs_ref, up_ref, dn_ref, out_ref):
    h = jnp.dot(xs_ref[...].astype(jnp.bfloat16),
                up_ref[0].astype(jnp.bfloat16).T,
                preferred_element_type=jnp.float32)
    g = _gelu(h)
    out_ref[...] = jnp.dot(g.astype(jnp.bfloat16),
                           dn_ref[0].astype(jnp.bfloat16).T,
                           preferred_element_type=jnp.float32)


def _shared_body(x_ref, up_ref, dn_ref, out_ref, xb_ref):
    e = pl.program_id(0)
    c = pl.program_id(1)

    @pl.when((e == 0) & (c == 0))
    def _():
        out_ref[...] = x_ref[...]
        xb_ref[...] = x_ref[...].astype(jnp.bfloat16)

    h = jnp.dot(xb_ref[...], up_ref[0].astype(jnp.bfloat16).T,
                preferred_element_type=jnp.float32)
    g = _gelu(h)
    out_ref[...] += jnp.dot(g.astype(jnp.bfloat16),
                            dn_ref[0].astype(jnp.bfloat16).T,
                            preferred_element_type=jnp.float32)


def _combine_body(sout_hbm, ys_hbm, pos_hbm, tp_hbm, out_hbm,
                  posc_v, tpc_v, rows_v, sv_v, ov_v,
                  sem_g0, sem_g1, sem_s0, sem_s1, sem_o0, sem_o1):
    wid = lax.axis_index("s") * 2 + lax.axis_index("c")
    base = wid * TPW
    nj = TPW // 16
    sems_g = (sem_g0, sem_g1)
    sems_s = (sem_s0, sem_s1)
    sems_o = (sem_o0, sem_o1)

    pltpu.sync_copy(pos_hbm.at[pl.ds(wid * (TPW // 16), TPW // 16)], posc_v)
    pltpu.sync_copy(tp_hbm.at[pl.ds(base * K, TPW * K)], tpc_v)

    def start_in(j):
        b = j % 2
        g = pltpu.async_copy(ys_hbm.at[posc_v.at[j]],
                             rows_v.at[b], sems_g[b])
        s = pltpu.async_copy(sout_hbm.at[pl.ds(base + 16 * j, 16)],
                             sv_v.at[b], sems_s[b])
        return g, s

    pend = start_in(0)
    out_pend = [None, None]
    for j in range(nj):
        b = j % 2
        g, s = pend
        g.wait()
        s.wait()
        if j + 1 < nj:
            pend = start_in(j + 1)

        def tok(jt, _):
            p0 = plsc.load_gather(tpc_v, [jnp.full((16,), 32 * j, jnp.int32) + 2 * jt])
            p1 = plsc.load_gather(tpc_v, [jnp.full((16,), 32 * j + 1, jnp.int32) + 2 * jt])
            for cc in range(H // 16):
                cs = pl.ds(cc * 16, 16)
                ov_v[b, jt, cs] = (sv_v[b, jt, cs] + p0 * rows_v[b, 2 * jt, cs]
                                   + p1 * rows_v[b, 2 * jt + 1, cs])
            return 0

        lax.fori_loop(0, 16, tok, 0)
        if out_pend[b] is not None:
            out_pend[b].wait()
        out_pend[b] = pltpu.async_copy(ov_v.at[b],
                                       out_hbm.at[pl.ds(base + 16 * j, 16)],
                                       sems_o[b])
    for h in out_pend:
        if h is not None:
            h.wait()


def kernel(x, shared_up, shared_down, routed_up, routed_down, router_w):
    rw_pad = jnp.zeros((LANES, H), jnp.float32).at[:NR].set(router_w)

    ti_wide, tp_wide = pl.pallas_call(
        _router_body,
        grid=(T // 512,),
        in_specs=[
            pl.BlockSpec((512, H), lambda t: (t, 0)),
            pl.BlockSpec((LANES, H), lambda t: (0, 0)),
        ],
        out_specs=[
            pl.BlockSpec((512, LANES), lambda t: (t, 0)),
            pl.BlockSpec((512, LANES), lambda t: (t, 0)),
        ],
        out_shape=[
            jax.ShapeDtypeStruct((T, LANES), jnp.int32),
            jax.ShapeDtypeStruct((T, LANES), jnp.float32),
        ],
    )(x, rw_pad)
    eid = ti_wide[:, :K].reshape(S)
    tp = tp_wide[:, :K].reshape(S)

    mesh = plsc.VectorSubcoreMesh(core_axis_name="c", subcore_axis_name="s")
    xs, pos, te = pl.kernel(
        _dispatch_body,
        out_type=[
            jax.ShapeDtypeStruct((P, H), jnp.float32),
            jax.ShapeDtypeStruct((S,), jnp.int32),
            jax.ShapeDtypeStruct((48,), jnp.int32),
        ],
        mesh=mesh,
        compiler_params=pltpu.CompilerParams(needs_layout_passes=False),
        scratch_types=[
            pltpu.VMEM((S,), jnp.int32),
            pltpu.VMEM((S,), jnp.int32),
            pltpu.VMEM((S,), jnp.int32),
            pltpu.VMEM((16,), jnp.int32),
            pltpu.VMEM((48,), jnp.int32),
            pltpu.VMEM((TPW,), jnp.int32),
            pltpu.VMEM((TPW,), jnp.int32),
            pltpu.VMEM((TPW, H), jnp.float32),
            pltpu.SemaphoreType.DMA,
        ],
    )(eid, x)

    ys = pl.pallas_call(
        _routed_body,
        grid_spec=pltpu.PrefetchScalarGridSpec(
            num_scalar_prefetch=1,
            grid=(EC, NT),
            in_specs=[
                pl.BlockSpec((BM, H), lambda c, i, te_r: (i, 0)),
                pl.BlockSpec((1, BE, H), lambda c, i, te_r: (te_r[i], c, 0)),
                pl.BlockSpec((1, H, BE), lambda c, i, te_r: (te_r[i], 0, c)),
            ],
            out_specs=pl.BlockSpec((BM, H), lambda c, i, te_r: (i, 0)),
        ),
        out_shape=jax.ShapeDtypeStruct((P, H), jnp.float32),
        compiler_params=pltpu.CompilerParams(
            dimension_semantics=("arbitrary", "arbitrary"),
        ),
    )(te[:NT], xs, routed_up, routed_down)

    s_out = pl.pallas_call(
        _shared_body,
        grid=(NS, E // 1024),
        in_specs=[
            pl.BlockSpec((T, H), lambda e, c: (0, 0)),
            pl.BlockSpec((1, 1024, H), lambda e, c: (e, c, 0)),
            pl.BlockSpec((1, H, 1024), lambda e, c: (e, 0, c)),
        ],
        out_specs=pl.BlockSpec((T, H), lambda e, c: (0, 0)),
        out_shape=jax.ShapeDtypeStruct((T, H), jnp.float32),
        scratch_shapes=[pltpu.VMEM((T, H), jnp.bfloat16)],
        compiler_params=pltpu.CompilerParams(
            dimension_semantics=("arbitrary", "arbitrary"),
        ),
    )(x, shared_up, shared_down)

    out = pl.kernel(
        _combine_body,
        out_type=jax.ShapeDtypeStruct((T, H), jnp.float32),
        mesh=mesh,
        compiler_params=pltpu.CompilerParams(needs_layout_passes=False),
        scratch_types=[
            pltpu.VMEM((TPW // 16, 32), jnp.int32),
            pltpu.VMEM((TPW * K,), jnp.float32),
            pltpu.VMEM((2, 32, H), jnp.float32),
            pltpu.VMEM((2, 16, H), jnp.float32),
            pltpu.VMEM((2, 16, H), jnp.float32),
            pltpu.SemaphoreType.DMA,
            pltpu.SemaphoreType.DMA,
            pltpu.SemaphoreType.DMA,
            pltpu.SemaphoreType.DMA,
            pltpu.SemaphoreType.DMA,
            pltpu.SemaphoreType.DMA,
        ],
    )(s_out, ys, pos.reshape(S // 32, 32), tp)
    return out
